# baseline (speedup 1.0000x reference)
import jax
import jax.numpy as jnp
from jax.experimental import pallas as pl
from jax.experimental.pallas import tpu as pltpu

_BN = 2048
_BB = 512


def _kern(x_ref, g_ref, c_ref, o_ref, kt_ref, cb_ref, ct_ref, sem):
    i = pl.program_id(0)
    j = pl.program_id(1)
    nb = g_ref.shape[0] // _BB
    offs = [0]
    for t in range(nb):
        offs.append(offs[-1] + (nb - t) * _BB)

    def _panel_copy(jj):
        lo = jj * _BB
        height = (nb - jj) * _BB
        return pltpu.make_async_copy(
            c_ref.at[pl.ds(lo, height), pl.ds(lo, _BB)],
            ct_ref.at[pl.ds(0, height), :],
            sem,
        )

    @pl.when(j == 0)
    def _compute_kt():
        @pl.when(i == 0)
        def _start_first_copy():
            _panel_copy(0).start()

        x = x_ref[...]
        xx = jnp.sum(x * x, axis=1, keepdims=True)
        for k in range(nb):
            g = g_ref[k * _BB:(k + 1) * _BB, :]
            gg = jnp.sum(g * g, axis=1)
            xg = jax.lax.dot_general(x, g, (((1,), (1,)), ((), ())),
                                     preferred_element_type=jnp.float32)
            sq = jnp.maximum(xx - 2.0 * xg + gg[None, :], 0.0)
            kt_ref[:, k * _BB:(k + 1) * _BB] = (
                jnp.exp(-0.5 * sq).astype(jnp.bfloat16))

    for jj in range(nb):
        @pl.when(j == jj)
        def _panel(jj=jj):
            lo = jj * _BB
            height = (nb - jj) * _BB
            off = offs[jj]

            @pl.when(i == 0)
            def _land_panel():
                _panel_copy(jj).wait()
                cb_ref[off:off + height, :] = (
                    ct_ref[0:height, :].astype(jnp.bfloat16))
                if jj + 1 < nb:
                    _panel_copy(jj + 1).start()

            o_ref[...] = jnp.dot(
                kt_ref[:, lo:],
                cb_ref[off:off + height, :],
                preferred_element_type=jnp.float32,
            )


def kernel(input, sparse_grid, chol_inv):
    n, d = input.shape
    m = sparse_grid.shape[0]
    nb = m // _BB
    tri_rows = sum((nb - t) * _BB for t in range(nb))

    return pl.pallas_call(
        _kern,
        grid=(n // _BN, nb),
        in_specs=[
            pl.BlockSpec((_BN, d), lambda i, j: (i, 0)),
            pl.BlockSpec((m, d), lambda i, j: (0, 0)),
            pl.BlockSpec(memory_space=pltpu.MemorySpace.HBM),
        ],
        out_specs=pl.BlockSpec((_BN, _BB), lambda i, j: (i, j)),
        out_shape=jax.ShapeDtypeStruct((n, m), jnp.float32),
        scratch_shapes=[
            pltpu.VMEM((_BN, m), jnp.bfloat16),
            pltpu.VMEM((tri_rows, _BB), jnp.bfloat16),
            pltpu.VMEM((m, _BB), jnp.float32),
            pltpu.SemaphoreType.DMA,
        ],
    )(input, sparse_grid, chol_inv)


# speedup vs baseline: 4.2428x; 4.2428x over previous
import jax
import jax.numpy as jnp
from jax.experimental import pallas as pl
from jax.experimental.pallas import tpu as pltpu

_BN = 1024
_BB = 1024
_BE = 512


def _kern(x_ref, g_ref, c_ref, o_ref, kt_ref, cb_ref, ct_ref, sem):
    i = pl.program_id(0)
    j = pl.program_id(1)
    m = g_ref.shape[0]
    nb = m // _BB
    offs = [0]
    for t in range(nb):
        offs.append(offs[-1] + (nb - t) * _BB)

    def _panel_copy(jj, half):
        lo = jj * _BB
        height = (nb - jj) * _BB
        return pltpu.make_async_copy(
            c_ref.at[pl.ds(lo, height), pl.ds(lo + half * _BE, _BE)],
            ct_ref.at[pl.ds(0, height), :],
            sem,
        )

    @pl.when(j == 0)
    def _compute_kt():
        @pl.when(i == 0)
        def _start_first_copy():
            _panel_copy(0, 0).start()

        x = x_ref[...]
        xx = jnp.sum(x * x, axis=1, keepdims=True)
        for k in range(m // _BE):
            g = g_ref[k * _BE:(k + 1) * _BE, :]
            gg = jnp.sum(g * g, axis=1)
            xg = jax.lax.dot_general(x, g, (((1,), (1,)), ((), ())),
                                     preferred_element_type=jnp.float32)
            sq = jnp.maximum(xx - 2.0 * xg + gg[None, :], 0.0)
            kt_ref[:, k * _BE:(k + 1) * _BE] = (
                jnp.exp(-0.5 * sq).astype(jnp.bfloat16))

    for jj in range(nb):
        @pl.when(j == jj)
        def _panel(jj=jj):
            lo = jj * _BB
            height = (nb - jj) * _BB
            off = offs[jj]

            @pl.when(i == 0)
            def _land_panel():
                _panel_copy(jj, 0).wait()
                cb_ref[off:off + height, 0:_BE] = (
                    ct_ref[0:height, :].astype(jnp.bfloat16))
                _panel_copy(jj, 1).start()
                _panel_copy(jj, 1).wait()
                cb_ref[off:off + height, _BE:] = (
                    ct_ref[0:height, :].astype(jnp.bfloat16))
                if jj + 1 < nb:
                    _panel_copy(jj + 1, 0).start()

            o_ref[...] = jnp.dot(
                kt_ref[:, lo:],
                cb_ref[off:off + height, :],
                preferred_element_type=jnp.float32,
            )


def kernel(input, sparse_grid, chol_inv):
    n, d = input.shape
    m = sparse_grid.shape[0]
    nb = m // _BB
    tri_rows = sum((nb - t) * _BB for t in range(nb))

    return pl.pallas_call(
        _kern,
        grid=(n // _BN, nb),
        in_specs=[
            pl.BlockSpec((_BN, d), lambda i, j: (i, 0)),
            pl.BlockSpec((m, d), lambda i, j: (0, 0)),
            pl.BlockSpec(memory_space=pltpu.MemorySpace.HBM),
        ],
        out_specs=pl.BlockSpec((_BN, _BB), lambda i, j: (i, j)),
        out_shape=jax.ShapeDtypeStruct((n, m), jnp.float32),
        compiler_params=pltpu.CompilerParams(
            vmem_limit_bytes=62 * 1024 * 1024),
        scratch_shapes=[
            pltpu.VMEM((_BN, m), jnp.bfloat16),
            pltpu.VMEM((tri_rows, _BB), jnp.bfloat16),
            pltpu.VMEM((m, _BE), jnp.float32),
            pltpu.SemaphoreType.DMA,
        ],
    )(input, sparse_grid, chol_inv)


# speedup vs baseline: 4.7621x; 1.1224x over previous
import jax
import jax.numpy as jnp
from jax.experimental import pallas as pl
from jax.experimental.pallas import tpu as pltpu

_BN = 1024
_BB = 512


def _kern(x_ref, g_ref, c_ref, o_ref, kt_ref, cb_ref):
    i = pl.program_id(0)
    j = pl.program_id(1)
    nb = g_ref.shape[0] // _BB
    offs = [0]
    for t in range(nb):
        offs.append(offs[-1] + (nb - t) * _BB)

    @pl.when(j == 0)
    def _compute_kt():
        x = x_ref[...]
        bx = -0.5 * jnp.sum(x * x, axis=1, keepdims=True)
        for k in range(nb):
            g = g_ref[k * _BB:(k + 1) * _BB, :]
            bg = -0.5 * jnp.sum(g * g, axis=1)
            xg = jax.lax.dot_general(x, g, (((1,), (1,)), ((), ())),
                                     preferred_element_type=jnp.float32)
            kt_ref[:, k * _BB:(k + 1) * _BB] = (
                jnp.exp((xg + bg[None, :]) + bx).astype(jnp.bfloat16))

    for jj in range(nb):
        @pl.when(j == jj)
        def _panel(jj=jj):
            lo = jj * _BB
            height = (nb - jj) * _BB
            off = offs[jj]

            @pl.when(i == 0)
            def _cast_panel():
                cb_ref[off:off + height, :] = (
                    c_ref[lo:, :].astype(jnp.bfloat16))

            o_ref[...] = jnp.dot(
                kt_ref[:, lo:],
                cb_ref[off:off + height, :],
                preferred_element_type=jnp.float32,
            )


def kernel(input, sparse_grid, chol_inv):
    n, d = input.shape
    m = sparse_grid.shape[0]
    nb = m // _BB
    tri_rows = sum((nb - t) * _BB for t in range(nb))

    return pl.pallas_call(
        _kern,
        grid=(n // _BN, nb),
        in_specs=[
            pl.BlockSpec((_BN, d), lambda i, j: (i, 0)),
            pl.BlockSpec((m, d), lambda i, j: (0, 0)),
            pl.BlockSpec((m, _BB),
                         lambda i, j: (0, jnp.where(i == 0, j, nb - 1))),
        ],
        out_specs=pl.BlockSpec((_BN, _BB), lambda i, j: (i, j)),
        out_shape=jax.ShapeDtypeStruct((n, m), jnp.float32),
        scratch_shapes=[
            pltpu.VMEM((_BN, m), jnp.bfloat16),
            pltpu.VMEM((tri_rows, _BB), jnp.bfloat16),
        ],
    )(input, sparse_grid, chol_inv)
